# baseline (speedup 1.0000x reference)
import functools

import jax
import jax.numpy as jnp
from jax import lax
from jax.experimental import pallas as pl
from jax.experimental.pallas import tpu as pltpu
from jax.experimental.pallas import tpu_sc as plsc

N = 10000
E = 320000
NC = 2
NS = 16
NW = NC * NS
BATCH = 128
NB = 80
EPW = NB * BATCH
EP = NW * EPW
NPAD = 10240
ROWS_PER_TILE = NPAD // NS

_MESH = plsc.VectorSubcoreMesh(core_axis_name="c", subcore_axis_name="s")
_SC_PARAMS = pltpu.CompilerParams(needs_layout_passes=False,
                                  use_tc_tiling_on_sc=False)


def _full16(val):
    return jnp.zeros((16,), jnp.int32) + val


def _rsqrt_f32(x):
    i = lax.bitcast_convert_type(x, jnp.int32)
    i = jnp.int32(0x5F3759DF) - lax.shift_right_logical(i, 1)
    y = lax.bitcast_convert_type(i, jnp.float32)
    for _ in range(3):
        y = y * (1.5 - 0.5 * x * y * y)
    return y


def _norm_sc(srcp, dstp, eap):
    def body(src_hbm, dst_hbm, ea_hbm, norm_hbm,
             src_v, dst_v, ea_v, ew_v, dinv_v, stripe_v,
             deg_sh, dinv_sh, sem):
        cid = lax.axis_index("c")
        sid = lax.axis_index("s")
        wid = cid * NS + sid

        spt = NPAD // NS
        def zero_body(i, _):
            stripe_v[pl.ds(i * 16, 16)] = jnp.zeros((16,), jnp.float32)
            return 0
        lax.fori_loop(0, spt // 16, zero_body, 0)
        pltpu.sync_copy(stripe_v, deg_sh.at[pl.ds(sid * spt, spt)])
        plsc.subcore_barrier()

        def deg_chunk(chunk):
            pltpu.sync_copy(src_hbm.at[chunk], src_v)
            pltpu.sync_copy(dst_hbm.at[chunk], dst_v)
            pltpu.sync_copy(ea_hbm.at[chunk], ea_v)

            def row_body(j, _):
                for q in range(BATCH // 16):
                    sl = pl.ds(q * 16, 16)
                    s = src_v[j, sl]
                    d = dst_v[j, sl]
                    a = ea_v[j, sl]
                    ew_v[j, sl] = jnp.where(s == d, 0.0, a)
                return 0
            lax.fori_loop(0, NB, row_body, 0)

            def scat_body(j, _):
                pltpu.sync_copy(ew_v.at[j], deg_sh.at[src_v.at[j]], add=True)
                return 0
            lax.fori_loop(0, NB, scat_body, 0)

        deg_chunk(2 * sid)
        deg_chunk(2 * sid + 1)
        plsc.subcore_barrier()

        pltpu.sync_copy(deg_sh.at[pl.ds(sid * spt, spt)], stripe_v)
        def dinv_body(i, _):
            sl = pl.ds(i * 16, 16)
            x = stripe_v[sl]
            stripe_v[sl] = jnp.where(x > 0.0, _rsqrt_f32(x), 0.0)
            return 0
        lax.fori_loop(0, spt // 16, dinv_body, 0)
        pltpu.sync_copy(stripe_v, dinv_sh.at[pl.ds(sid * spt, spt)])
        plsc.subcore_barrier()

        pltpu.sync_copy(dinv_sh, dinv_v)
        pltpu.sync_copy(src_hbm.at[wid], src_v)
        pltpu.sync_copy(dst_hbm.at[wid], dst_v)
        pltpu.sync_copy(ea_hbm.at[wid], ea_v)

        def norm_body(j, _):
            for q in range(BATCH // 16):
                sl = pl.ds(q * 16, 16)
                s = src_v[j, sl]
                d = dst_v[j, sl]
                a = ea_v[j, sl]
                ew = jnp.where(s == d, 0.0, a)
                ns = plsc.load_gather(dinv_v, [s])
                nd = plsc.load_gather(dinv_v, [d])
                ew_v[j, sl] = -(ns * ew) * nd
            return 0
        lax.fori_loop(0, NB, norm_body, 0)
        pltpu.sync_copy(ew_v, norm_hbm.at[wid])

    f = pl.kernel(
        body,
        out_type=jax.ShapeDtypeStruct((NW, NB, BATCH), jnp.float32),
        mesh=_MESH,
        scratch_types=[
            pltpu.VMEM((NB, BATCH), jnp.int32),
            pltpu.VMEM((NB, BATCH), jnp.int32),
            pltpu.VMEM((NB, BATCH), jnp.float32),
            pltpu.VMEM((NB, BATCH), jnp.float32),
            pltpu.VMEM((NPAD,), jnp.float32),
            pltpu.VMEM((NPAD // NS,), jnp.float32),
            pltpu.VMEM_SHARED((NPAD,), jnp.float32),
            pltpu.VMEM_SHARED((NPAD,), jnp.float32),
            pltpu.SemaphoreType.DMA,
        ],
        compiler_params=_SC_PARAMS,
        name="cheby_norm_sc",
    )
    return f(srcp, dstp, eap)


_NBUF = 4


def _prop_sc(y, srcp, dstp, normp, width):
    def body(y_hbm, src_hbm, dst_hbm, nrm_hbm, out_hbm,
             src_v, dst_v, nrm_v, rows0, rows1, rows2, rows3,
             acc_sh, gsem0, gsem1, gsem2, gsem3, ssem):
        cid = lax.axis_index("c")
        sid = lax.axis_index("s")
        wid = cid * NS + sid
        rows = (rows0, rows1, rows2, rows3)
        gsems = (gsem0, gsem1, gsem2, gsem3)

        pltpu.sync_copy(src_hbm.at[wid], src_v)
        pltpu.sync_copy(dst_hbm.at[wid], dst_v)
        pltpu.sync_copy(nrm_hbm.at[wid], nrm_v)

        @plsc.parallel_loop(0, BATCH, step=1, unroll=8)
        def _(e):
            for q in range(width // 16):
                rows0[e, pl.ds(q * 16, 16)] = jnp.zeros((16,), jnp.float32)
        base = sid * ROWS_PER_TILE
        for off in range(0, ROWS_PER_TILE, BATCH):
            pltpu.sync_copy(rows0, acc_sh.at[pl.ds(base + off, BATCH)])
        plsc.subcore_barrier()

        def scale(buf, j):
            @plsc.parallel_loop(0, BATCH, step=1, unroll=8)
            def _(e):
                nsp = plsc.load_gather(nrm_v, [_full16(j), _full16(e)])
                for q in range(width // 16):
                    sl = pl.ds(q * 16, 16)
                    buf[e, sl] = buf[e, sl] * nsp

        def grp(g, _):
            j0 = g * _NBUF
            gs = [pltpu.async_copy(y_hbm.at[src_v.at[j0 + b]], rows[b],
                                   gsems[b])
                  for b in range(_NBUF)]
            ss = []
            for b in range(_NBUF):
                gs[b].wait()
                scale(rows[b], j0 + b)
                ss.append(pltpu.async_copy(rows[b],
                                           acc_sh.at[dst_v.at[j0 + b]],
                                           ssem, add=True))
            for s in ss:
                s.wait()
            return 0
        lax.fori_loop(0, NB // _NBUF, grp, 0)
        plsc.subcore_barrier()

        pltpu.sync_copy(acc_sh.at[pl.ds(base, ROWS_PER_TILE)],
                        out_hbm.at[cid, pl.ds(base, ROWS_PER_TILE)])

    f = pl.kernel(
        body,
        out_type=jax.ShapeDtypeStruct((NC, NPAD, width), jnp.float32),
        mesh=_MESH,
        scratch_types=[
            pltpu.VMEM((NB, BATCH), jnp.int32),
            pltpu.VMEM((NB, BATCH), jnp.int32),
            pltpu.VMEM((NB, BATCH), jnp.float32),
            pltpu.VMEM((BATCH, width), jnp.float32),
            pltpu.VMEM((BATCH, width), jnp.float32),
            pltpu.VMEM((BATCH, width), jnp.float32),
            pltpu.VMEM((BATCH, width), jnp.float32),
            pltpu.VMEM_SHARED((NPAD, width), jnp.float32),
            pltpu.SemaphoreType.DMA,
            pltpu.SemaphoreType.DMA,
            pltpu.SemaphoreType.DMA,
            pltpu.SemaphoreType.DMA,
            pltpu.SemaphoreType.DMA,
        ],
        compiler_params=_SC_PARAMS,
        name=f"cheby_prop_sc_{width}",
    )
    return f(y, srcp, dstp, normp)


_BLK = 1024


def _mm_tc(h, wcat):
    din = h.shape[1]
    wd = wcat.shape[1] // 3

    def body(h_ref, w_ref, o0, o1, o2):
        acc = jnp.dot(h_ref[...], w_ref[...], preferred_element_type=jnp.float32)
        o0[...] = acc[:, :wd]
        o1[...] = acc[:, wd:2 * wd]
        o2[...] = acc[:, 2 * wd:]

    outs = [jax.ShapeDtypeStruct((NPAD, wd), jnp.float32)] * 3
    return pl.pallas_call(
        body,
        grid=(NPAD // _BLK,),
        in_specs=[
            pl.BlockSpec((_BLK, din), lambda i: (i, 0)),
            pl.BlockSpec((din, 3 * wd), lambda i: (0, 0)),
        ],
        out_specs=[pl.BlockSpec((_BLK, wd), lambda i: (i, 0))] * 3,
        out_shape=outs,
    )(h, wcat)


def _comb_tc(p1, la):
    wd = p1.shape[1]

    def body(p1_ref, la0_ref, la1_ref, o_ref):
        o_ref[...] = p1_ref[...] + 2.0 * (la0_ref[0] + la1_ref[0])

    return pl.pallas_call(
        body,
        grid=(NPAD // _BLK,),
        in_specs=[
            pl.BlockSpec((_BLK, wd), lambda i: (i, 0)),
            pl.BlockSpec((1, _BLK, wd), lambda i: (0, i, 0)),
            pl.BlockSpec((1, _BLK, wd), lambda i: (1, i, 0)),
        ],
        out_specs=pl.BlockSpec((_BLK, wd), lambda i: (i, 0)),
        out_shape=jax.ShapeDtypeStruct((NPAD, wd), jnp.float32),
    )(p1, la, la)


def _end_mm_tc(p0, p2, lb, b, wcat):
    wd = p0.shape[1]
    wo = wcat.shape[1] // 3

    def body(p0_ref, p2_ref, lb0_ref, lb1_ref, b_ref, w_ref, o0, o1, o2):
        h = p0_ref[...] - p2_ref[...] + lb0_ref[0] + lb1_ref[0] + b_ref[...]
        h = jnp.maximum(h, 0.0)
        acc = jnp.dot(h, w_ref[...], preferred_element_type=jnp.float32)
        o0[...] = acc[:, :wo]
        o1[...] = acc[:, wo:2 * wo]
        o2[...] = acc[:, 2 * wo:]

    outs = [jax.ShapeDtypeStruct((NPAD, wo), jnp.float32)] * 3
    return pl.pallas_call(
        body,
        grid=(NPAD // _BLK,),
        in_specs=[
            pl.BlockSpec((_BLK, wd), lambda i: (i, 0)),
            pl.BlockSpec((_BLK, wd), lambda i: (i, 0)),
            pl.BlockSpec((1, _BLK, wd), lambda i: (0, i, 0)),
            pl.BlockSpec((1, _BLK, wd), lambda i: (1, i, 0)),
            pl.BlockSpec((1, wd), lambda i: (0, 0)),
            pl.BlockSpec((wd, 3 * wo), lambda i: (0, 0)),
        ],
        out_specs=[pl.BlockSpec((_BLK, wo), lambda i: (i, 0))] * 3,
        out_shape=outs,
    )(p0, p2, lb, lb, b.reshape(1, wd), wcat)


def _final_tc(p0, p2, lb, b):
    wd = p0.shape[1]

    def body(p0_ref, p2_ref, lb0_ref, lb1_ref, b_ref, o_ref):
        z = p0_ref[...] - p2_ref[...] + lb0_ref[0] + lb1_ref[0] + b_ref[...]
        z = jnp.maximum(z, 0.0)
        m = jnp.max(z, axis=-1, keepdims=True)
        zs = z - m
        lse = jnp.log(jnp.sum(jnp.exp(zs), axis=-1, keepdims=True))
        o_ref[...] = zs - lse

    return pl.pallas_call(
        body,
        grid=(NPAD // _BLK,),
        in_specs=[
            pl.BlockSpec((_BLK, wd), lambda i: (i, 0)),
            pl.BlockSpec((_BLK, wd), lambda i: (i, 0)),
            pl.BlockSpec((1, _BLK, wd), lambda i: (0, i, 0)),
            pl.BlockSpec((1, _BLK, wd), lambda i: (1, i, 0)),
            pl.BlockSpec((1, wd), lambda i: (0, 0)),
        ],
        out_specs=pl.BlockSpec((_BLK, wd), lambda i: (i, 0)),
        out_shape=jax.ShapeDtypeStruct((NPAD, wd), jnp.float32),
    )(p0, p2, lb, lb, b.reshape(1, wd))


def kernel(x, edge_index, edge_attr, W0, b0, W1, b1, W2, b2):
    src = edge_index[0]
    dst = edge_index[1]

    pad = EP - E
    srcp = jnp.concatenate([src, jnp.zeros((pad,), jnp.int32)]).reshape(NW, NB, BATCH)
    dstp = jnp.concatenate([dst, jnp.zeros((pad,), jnp.int32)]).reshape(NW, NB, BATCH)
    eap = jnp.concatenate([edge_attr, jnp.zeros((pad,), jnp.float32)]).reshape(NW, NB, BATCH)

    normp = _norm_sc(srcp, dstp, eap)

    wc1 = jnp.concatenate([W0[0], W0[1], W0[2]], axis=1)
    wc2 = jnp.concatenate([W1[0], W1[1], W1[2]], axis=1)
    wc3 = jnp.concatenate([W2[0], W2[1], W2[2]], axis=1)

    p0, p1, p2 = _mm_tc(x, wc1)
    la = _prop_sc(p2, srcp, dstp, normp, 64)
    bmat = _comb_tc(p1, la)
    lb = _prop_sc(bmat, srcp, dstp, normp, 64)
    q0, q1, q2 = _end_mm_tc(p0, p2, lb, b0, wc2)
    la = _prop_sc(q2, srcp, dstp, normp, 64)
    bmat = _comb_tc(q1, la)
    lb = _prop_sc(bmat, srcp, dstp, normp, 64)
    r0, r1, r2 = _end_mm_tc(q0, q2, lb, b1, wc3)
    la = _prop_sc(r2, srcp, dstp, normp, 32)
    bmat = _comb_tc(r1, la)
    lb = _prop_sc(bmat, srcp, dstp, normp, 32)

    return _final_tc(r0, r2, lb, b2)[:N]


# speedup vs baseline: 1.9655x; 1.9655x over previous
import functools

import jax
import jax.numpy as jnp
from jax import lax
from jax.experimental import pallas as pl
from jax.experimental.pallas import tpu as pltpu
from jax.experimental.pallas import tpu_sc as plsc

N = 10000
E = 320000
NC = 2
NS = 16
NW = NC * NS
BATCH = 128
NB = 80
EPW = NB * BATCH
EP = NW * EPW
NPAD = 10240
ROWS_PER_TILE = NPAD // NS

_MESH = plsc.VectorSubcoreMesh(core_axis_name="c", subcore_axis_name="s")
_SC_PARAMS = pltpu.CompilerParams(needs_layout_passes=False,
                                  use_tc_tiling_on_sc=False)


def _full16(val):
    return jnp.zeros((16,), jnp.int32) + val


def _rsqrt_f32(x):
    i = lax.bitcast_convert_type(x, jnp.int32)
    i = jnp.int32(0x5F3759DF) - lax.shift_right_logical(i, 1)
    y = lax.bitcast_convert_type(i, jnp.float32)
    for _ in range(3):
        y = y * (1.5 - 0.5 * x * y * y)
    return y


def _norm_sc(srcp, dstp, eap):
    def body(src_hbm, dst_hbm, ea_hbm, norm_hbm,
             src_v, dst_v, ea_v, ew_v, dinv_v, stripe_v,
             deg_sh, dinv_sh, sem):
        cid = lax.axis_index("c")
        sid = lax.axis_index("s")
        wid = cid * NS + sid

        spt = NPAD // NS
        def zero_body(i, _):
            stripe_v[pl.ds(i * 16, 16)] = jnp.zeros((16,), jnp.float32)
            return 0
        lax.fori_loop(0, spt // 16, zero_body, 0)
        pltpu.sync_copy(stripe_v, deg_sh.at[pl.ds(sid * spt, spt)])
        plsc.subcore_barrier()

        def deg_chunk(chunk):
            pltpu.sync_copy(src_hbm.at[chunk], src_v)
            pltpu.sync_copy(dst_hbm.at[chunk], dst_v)
            pltpu.sync_copy(ea_hbm.at[chunk], ea_v)

            def row_body(j, _):
                for q in range(BATCH // 16):
                    sl = pl.ds(q * 16, 16)
                    s = src_v[j, sl]
                    d = dst_v[j, sl]
                    a = ea_v[j, sl]
                    ew_v[j, sl] = jnp.where(s == d, 0.0, a)
                return 0
            lax.fori_loop(0, NB, row_body, 0)

            def scat_body(j, _):
                pltpu.sync_copy(ew_v.at[j], deg_sh.at[src_v.at[j]], add=True)
                return 0
            lax.fori_loop(0, NB, scat_body, 0)

        deg_chunk(2 * sid)
        deg_chunk(2 * sid + 1)
        plsc.subcore_barrier()

        pltpu.sync_copy(deg_sh.at[pl.ds(sid * spt, spt)], stripe_v)
        def dinv_body(i, _):
            sl = pl.ds(i * 16, 16)
            x = stripe_v[sl]
            stripe_v[sl] = jnp.where(x > 0.0, _rsqrt_f32(x), 0.0)
            return 0
        lax.fori_loop(0, spt // 16, dinv_body, 0)
        pltpu.sync_copy(stripe_v, dinv_sh.at[pl.ds(sid * spt, spt)])
        plsc.subcore_barrier()

        pltpu.sync_copy(dinv_sh, dinv_v)
        pltpu.sync_copy(src_hbm.at[wid], src_v)
        pltpu.sync_copy(dst_hbm.at[wid], dst_v)
        pltpu.sync_copy(ea_hbm.at[wid], ea_v)

        def norm_body(j, _):
            for q in range(BATCH // 16):
                sl = pl.ds(q * 16, 16)
                s = src_v[j, sl]
                d = dst_v[j, sl]
                a = ea_v[j, sl]
                ew = jnp.where(s == d, 0.0, a)
                ns = plsc.load_gather(dinv_v, [s])
                nd = plsc.load_gather(dinv_v, [d])
                ew_v[j, sl] = -(ns * ew) * nd
            return 0
        lax.fori_loop(0, NB, norm_body, 0)
        pltpu.sync_copy(ew_v, norm_hbm.at[wid])

    f = pl.kernel(
        body,
        out_type=jax.ShapeDtypeStruct((NW, NB, BATCH), jnp.float32),
        mesh=_MESH,
        scratch_types=[
            pltpu.VMEM((NB, BATCH), jnp.int32),
            pltpu.VMEM((NB, BATCH), jnp.int32),
            pltpu.VMEM((NB, BATCH), jnp.float32),
            pltpu.VMEM((NB, BATCH), jnp.float32),
            pltpu.VMEM((NPAD,), jnp.float32),
            pltpu.VMEM((NPAD // NS,), jnp.float32),
            pltpu.VMEM_SHARED((NPAD,), jnp.float32),
            pltpu.VMEM_SHARED((NPAD,), jnp.float32),
            pltpu.SemaphoreType.DMA,
        ],
        compiler_params=_SC_PARAMS,
        name="cheby_norm_sc",
    )
    return f(srcp, dstp, eap)


_NBUF = 2


def _prop_sc(y, srcp, dstp, normp, width):
    def body(y_hbm, src_hbm, dst_hbm, nrm_hbm, out_hbm,
             src_v, dst_v, nrm_v, rows0, rows1, rows2, rows3,
             y_sh, acc_sh, gsem0, gsem1, gsem2, gsem3,
             ssem0, ssem1, ssem2, ssem3):
        cid = lax.axis_index("c")
        sid = lax.axis_index("s")
        wid = cid * NS + sid
        rows = (rows0, rows1, rows2, rows3)
        gsems = (gsem0, gsem1, gsem2, gsem3)
        ssems = (ssem0, ssem1, ssem2, ssem3)

        pltpu.sync_copy(src_hbm.at[wid], src_v)
        pltpu.sync_copy(dst_hbm.at[wid], dst_v)
        pltpu.sync_copy(nrm_hbm.at[wid], nrm_v)

        stg = sid * ROWS_PER_TILE
        pltpu.sync_copy(y_hbm.at[pl.ds(stg, ROWS_PER_TILE)],
                        y_sh.at[pl.ds(stg, ROWS_PER_TILE)])

        @plsc.parallel_loop(0, BATCH, step=1, unroll=8)
        def _(e):
            for q in range(width // 16):
                rows0[e, pl.ds(q * 16, 16)] = jnp.zeros((16,), jnp.float32)
        base = sid * ROWS_PER_TILE
        for off in range(0, ROWS_PER_TILE, BATCH):
            pltpu.sync_copy(rows0, acc_sh.at[pl.ds(base + off, BATCH)])
        plsc.subcore_barrier()

        def scale(buf, j):
            @plsc.parallel_loop(0, BATCH, step=1, unroll=8)
            def _(e):
                nsp = plsc.load_gather(nrm_v, [_full16(j), _full16(e)])
                for q in range(width // 16):
                    sl = pl.ds(q * 16, 16)
                    buf[e, sl] = buf[e, sl] * nsp

        def grp(g, _):
            j0 = g * _NBUF
            gs = [pltpu.async_copy(y_sh.at[src_v.at[j0 + b]], rows[b],
                                   gsems[b])
                  for b in range(_NBUF)]
            ss = []
            for b in range(_NBUF):
                gs[b].wait()
                scale(rows[b], j0 + b)
                ss.append(pltpu.async_copy(rows[b],
                                           acc_sh.at[dst_v.at[j0 + b]],
                                           ssems[b], add=True))
            for s in ss:
                s.wait()
            return 0
        lax.fori_loop(0, NB // _NBUF, grp, 0)
        plsc.subcore_barrier()

        pltpu.sync_copy(acc_sh.at[pl.ds(base, ROWS_PER_TILE)],
                        out_hbm.at[cid, pl.ds(base, ROWS_PER_TILE)])

    f = pl.kernel(
        body,
        out_type=jax.ShapeDtypeStruct((NC, NPAD, width), jnp.float32),
        mesh=_MESH,
        scratch_types=[
            pltpu.VMEM((NB, BATCH), jnp.int32),
            pltpu.VMEM((NB, BATCH), jnp.int32),
            pltpu.VMEM((NB, BATCH), jnp.float32),
            pltpu.VMEM((BATCH, width), jnp.float32),
            pltpu.VMEM((BATCH, width), jnp.float32),
            pltpu.VMEM((BATCH, width), jnp.float32),
            pltpu.VMEM((BATCH, width), jnp.float32),
            pltpu.VMEM_SHARED((NPAD, width), jnp.float32),
            pltpu.VMEM_SHARED((NPAD, width), jnp.float32),
            pltpu.SemaphoreType.DMA,
            pltpu.SemaphoreType.DMA,
            pltpu.SemaphoreType.DMA,
            pltpu.SemaphoreType.DMA,
            pltpu.SemaphoreType.DMA,
            pltpu.SemaphoreType.DMA,
            pltpu.SemaphoreType.DMA,
            pltpu.SemaphoreType.DMA,
        ],
        compiler_params=_SC_PARAMS,
        name=f"cheby_prop_sc_{width}",
    )
    return f(y, srcp, dstp, normp)


_BLK = 1024


def _mm_tc(h, wcat):
    din = h.shape[1]
    wd = wcat.shape[1] // 3

    def body(h_ref, w_ref, o0, o1, o2):
        acc = jnp.dot(h_ref[...], w_ref[...], preferred_element_type=jnp.float32)
        o0[...] = acc[:, :wd]
        o1[...] = acc[:, wd:2 * wd]
        o2[...] = acc[:, 2 * wd:]

    outs = [jax.ShapeDtypeStruct((NPAD, wd), jnp.float32)] * 3
    return pl.pallas_call(
        body,
        grid=(NPAD // _BLK,),
        in_specs=[
            pl.BlockSpec((_BLK, din), lambda i: (i, 0)),
            pl.BlockSpec((din, 3 * wd), lambda i: (0, 0)),
        ],
        out_specs=[pl.BlockSpec((_BLK, wd), lambda i: (i, 0))] * 3,
        out_shape=outs,
    )(h, wcat)


def _comb_tc(p1, la):
    wd = p1.shape[1]

    def body(p1_ref, la0_ref, la1_ref, o_ref):
        o_ref[...] = p1_ref[...] + 2.0 * (la0_ref[0] + la1_ref[0])

    return pl.pallas_call(
        body,
        grid=(NPAD // _BLK,),
        in_specs=[
            pl.BlockSpec((_BLK, wd), lambda i: (i, 0)),
            pl.BlockSpec((1, _BLK, wd), lambda i: (0, i, 0)),
            pl.BlockSpec((1, _BLK, wd), lambda i: (1, i, 0)),
        ],
        out_specs=pl.BlockSpec((_BLK, wd), lambda i: (i, 0)),
        out_shape=jax.ShapeDtypeStruct((NPAD, wd), jnp.float32),
    )(p1, la, la)


def _end_mm_tc(p0, p2, lb, b, wcat):
    wd = p0.shape[1]
    wo = wcat.shape[1] // 3

    def body(p0_ref, p2_ref, lb0_ref, lb1_ref, b_ref, w_ref, o0, o1, o2):
        h = p0_ref[...] - p2_ref[...] + lb0_ref[0] + lb1_ref[0] + b_ref[...]
        h = jnp.maximum(h, 0.0)
        acc = jnp.dot(h, w_ref[...], preferred_element_type=jnp.float32)
        o0[...] = acc[:, :wo]
        o1[...] = acc[:, wo:2 * wo]
        o2[...] = acc[:, 2 * wo:]

    outs = [jax.ShapeDtypeStruct((NPAD, wo), jnp.float32)] * 3
    return pl.pallas_call(
        body,
        grid=(NPAD // _BLK,),
        in_specs=[
            pl.BlockSpec((_BLK, wd), lambda i: (i, 0)),
            pl.BlockSpec((_BLK, wd), lambda i: (i, 0)),
            pl.BlockSpec((1, _BLK, wd), lambda i: (0, i, 0)),
            pl.BlockSpec((1, _BLK, wd), lambda i: (1, i, 0)),
            pl.BlockSpec((1, wd), lambda i: (0, 0)),
            pl.BlockSpec((wd, 3 * wo), lambda i: (0, 0)),
        ],
        out_specs=[pl.BlockSpec((_BLK, wo), lambda i: (i, 0))] * 3,
        out_shape=outs,
    )(p0, p2, lb, lb, b.reshape(1, wd), wcat)


def _final_tc(p0, p2, lb, b):
    wd = p0.shape[1]

    def body(p0_ref, p2_ref, lb0_ref, lb1_ref, b_ref, o_ref):
        z = p0_ref[...] - p2_ref[...] + lb0_ref[0] + lb1_ref[0] + b_ref[...]
        z = jnp.maximum(z, 0.0)
        m = jnp.max(z, axis=-1, keepdims=True)
        zs = z - m
        lse = jnp.log(jnp.sum(jnp.exp(zs), axis=-1, keepdims=True))
        o_ref[...] = zs - lse

    return pl.pallas_call(
        body,
        grid=(NPAD // _BLK,),
        in_specs=[
            pl.BlockSpec((_BLK, wd), lambda i: (i, 0)),
            pl.BlockSpec((_BLK, wd), lambda i: (i, 0)),
            pl.BlockSpec((1, _BLK, wd), lambda i: (0, i, 0)),
            pl.BlockSpec((1, _BLK, wd), lambda i: (1, i, 0)),
            pl.BlockSpec((1, wd), lambda i: (0, 0)),
        ],
        out_specs=pl.BlockSpec((_BLK, wd), lambda i: (i, 0)),
        out_shape=jax.ShapeDtypeStruct((NPAD, wd), jnp.float32),
    )(p0, p2, lb, lb, b.reshape(1, wd))


def kernel(x, edge_index, edge_attr, W0, b0, W1, b1, W2, b2):
    src = edge_index[0]
    dst = edge_index[1]

    pad = EP - E
    srcp = jnp.concatenate([src, jnp.zeros((pad,), jnp.int32)]).reshape(NW, NB, BATCH)
    dstp = jnp.concatenate([dst, jnp.zeros((pad,), jnp.int32)]).reshape(NW, NB, BATCH)
    eap = jnp.concatenate([edge_attr, jnp.zeros((pad,), jnp.float32)]).reshape(NW, NB, BATCH)

    normp = _norm_sc(srcp, dstp, eap)

    wc1 = jnp.concatenate([W0[0], W0[1], W0[2]], axis=1)
    wc2 = jnp.concatenate([W1[0], W1[1], W1[2]], axis=1)
    wc3 = jnp.concatenate([W2[0], W2[1], W2[2]], axis=1)

    p0, p1, p2 = _mm_tc(x, wc1)
    la = _prop_sc(p2, srcp, dstp, normp, 64)
    bmat = _comb_tc(p1, la)
    lb = _prop_sc(bmat, srcp, dstp, normp, 64)
    q0, q1, q2 = _end_mm_tc(p0, p2, lb, b0, wc2)
    la = _prop_sc(q2, srcp, dstp, normp, 64)
    bmat = _comb_tc(q1, la)
    lb = _prop_sc(bmat, srcp, dstp, normp, 64)
    r0, r1, r2 = _end_mm_tc(q0, q2, lb, b1, wc3)
    la = _prop_sc(r2, srcp, dstp, normp, 32)
    bmat = _comb_tc(r1, la)
    lb = _prop_sc(bmat, srcp, dstp, normp, 32)

    return _final_tc(r0, r2, lb, b2)[:N]


# speedup vs baseline: 2.1584x; 1.0981x over previous
import functools

import jax
import jax.numpy as jnp
from jax import lax
from jax.experimental import pallas as pl
from jax.experimental.pallas import tpu as pltpu
from jax.experimental.pallas import tpu_sc as plsc

N = 10000
E = 320000
NC = 2
NS = 16
NW = NC * NS
BATCH = 128
NB = 80
EPW = NB * BATCH
EP = NW * EPW
NPAD = 10240
ROWS_PER_TILE = NPAD // NS

_MESH = plsc.VectorSubcoreMesh(core_axis_name="c", subcore_axis_name="s")
_SC_PARAMS = pltpu.CompilerParams(needs_layout_passes=False,
                                  use_tc_tiling_on_sc=False)


def _full16(val):
    return jnp.zeros((16,), jnp.int32) + val


def _rsqrt_f32(x):
    i = lax.bitcast_convert_type(x, jnp.int32)
    i = jnp.int32(0x5F3759DF) - lax.shift_right_logical(i, 1)
    y = lax.bitcast_convert_type(i, jnp.float32)
    for _ in range(3):
        y = y * (1.5 - 0.5 * x * y * y)
    return y


def _norm_sc(srcp, dstp, eap):
    def body(src_hbm, dst_hbm, ea_hbm, norm_hbm,
             src_v, dst_v, ea_v, ew_v, dinv_v, stripe_v,
             deg_sh, dinv_sh, sem):
        cid = lax.axis_index("c")
        sid = lax.axis_index("s")
        wid = cid * NS + sid

        spt = NPAD // NS
        def zero_body(i, _):
            stripe_v[pl.ds(i * 16, 16)] = jnp.zeros((16,), jnp.float32)
            return 0
        lax.fori_loop(0, spt // 16, zero_body, 0)
        pltpu.sync_copy(stripe_v, deg_sh.at[pl.ds(sid * spt, spt)])
        plsc.subcore_barrier()

        def deg_chunk(chunk):
            pltpu.sync_copy(src_hbm.at[chunk], src_v)
            pltpu.sync_copy(dst_hbm.at[chunk], dst_v)
            pltpu.sync_copy(ea_hbm.at[chunk], ea_v)

            def row_body(j, _):
                for q in range(BATCH // 16):
                    sl = pl.ds(q * 16, 16)
                    s = src_v[j, sl]
                    d = dst_v[j, sl]
                    a = ea_v[j, sl]
                    ew_v[j, sl] = jnp.where(s == d, 0.0, a)
                return 0
            lax.fori_loop(0, NB, row_body, 0)

            def scat_body(j, _):
                pltpu.sync_copy(ew_v.at[j], deg_sh.at[src_v.at[j]], add=True)
                return 0
            lax.fori_loop(0, NB, scat_body, 0)

        deg_chunk(2 * sid)
        deg_chunk(2 * sid + 1)
        plsc.subcore_barrier()

        pltpu.sync_copy(deg_sh.at[pl.ds(sid * spt, spt)], stripe_v)
        def dinv_body(i, _):
            sl = pl.ds(i * 16, 16)
            x = stripe_v[sl]
            stripe_v[sl] = jnp.where(x > 0.0, _rsqrt_f32(x), 0.0)
            return 0
        lax.fori_loop(0, spt // 16, dinv_body, 0)
        pltpu.sync_copy(stripe_v, dinv_sh.at[pl.ds(sid * spt, spt)])
        plsc.subcore_barrier()

        pltpu.sync_copy(dinv_sh, dinv_v)
        pltpu.sync_copy(src_hbm.at[wid], src_v)
        pltpu.sync_copy(dst_hbm.at[wid], dst_v)
        pltpu.sync_copy(ea_hbm.at[wid], ea_v)

        def norm_body(j, _):
            for q in range(BATCH // 16):
                sl = pl.ds(q * 16, 16)
                s = src_v[j, sl]
                d = dst_v[j, sl]
                a = ea_v[j, sl]
                ew = jnp.where(s == d, 0.0, a)
                ns = plsc.load_gather(dinv_v, [s])
                nd = plsc.load_gather(dinv_v, [d])
                ew_v[j, sl] = -(ns * ew) * nd
            return 0
        lax.fori_loop(0, NB, norm_body, 0)
        pltpu.sync_copy(ew_v, norm_hbm.at[wid])

    f = pl.kernel(
        body,
        out_type=jax.ShapeDtypeStruct((NW, NB, BATCH), jnp.float32),
        mesh=_MESH,
        scratch_types=[
            pltpu.VMEM((NB, BATCH), jnp.int32),
            pltpu.VMEM((NB, BATCH), jnp.int32),
            pltpu.VMEM((NB, BATCH), jnp.float32),
            pltpu.VMEM((NB, BATCH), jnp.float32),
            pltpu.VMEM((NPAD,), jnp.float32),
            pltpu.VMEM((NPAD // NS,), jnp.float32),
            pltpu.VMEM_SHARED((NPAD,), jnp.float32),
            pltpu.VMEM_SHARED((NPAD,), jnp.float32),
            pltpu.SemaphoreType.DMA,
        ],
        compiler_params=_SC_PARAMS,
        name="cheby_norm_sc",
    )
    return f(srcp, dstp, eap)


_NBUF = 4


def _prop_sc(y, srcp, dstp, normp, width):
    G = NB // _NBUF

    def body(y_hbm, src_hbm, dst_hbm, nrm_hbm, out_hbm,
             srcb, dstb, nrmb, rows0, rows1, rows2, rows3,
             y_sh, acc_sh, gsem0, gsem1, gsem2, gsem3,
             ssem0, ssem1, ssem2, ssem3, isem):
        cid = lax.axis_index("c")
        sid = lax.axis_index("s")
        wid = cid * NS + sid
        rows = (rows0, rows1, rows2, rows3)
        gsems = (gsem0, gsem1, gsem2, gsem3)
        ssems = (ssem0, ssem1, ssem2, ssem3)
        streams = ((src_hbm, srcb), (dst_hbm, dstb), (nrm_hbm, nrmb))

        def idx_fire(g, p):
            for hbm, vb in streams:
                pltpu.async_copy(hbm.at[wid, pl.ds(g * _NBUF, _NBUF)],
                                 vb.at[pl.ds(p * _NBUF, _NBUF)], isem)

        def idx_wait(g, p):
            for hbm, vb in streams:
                pltpu.make_async_copy(
                    hbm.at[wid, pl.ds(g * _NBUF, _NBUF)],
                    vb.at[pl.ds(p * _NBUF, _NBUF)], isem).wait()

        idx_fire(0, 0)

        stg = sid * ROWS_PER_TILE
        pltpu.sync_copy(y_hbm.at[pl.ds(stg, ROWS_PER_TILE)],
                        y_sh.at[pl.ds(stg, ROWS_PER_TILE)])

        @plsc.parallel_loop(0, BATCH, step=1, unroll=8)
        def _(e):
            for q in range(width // 16):
                rows0[e, pl.ds(q * 16, 16)] = jnp.zeros((16,), jnp.float32)
        base = sid * ROWS_PER_TILE
        for off in range(0, ROWS_PER_TILE, BATCH):
            pltpu.sync_copy(rows0, acc_sh.at[pl.ds(base + off, BATCH)])
        plsc.subcore_barrier()

        def scale(buf, r):
            @plsc.parallel_loop(0, BATCH, step=1, unroll=8)
            def _(e):
                nsp = plsc.load_gather(nrmb, [_full16(r), _full16(e)])
                for q in range(width // 16):
                    sl = pl.ds(q * 16, 16)
                    buf[e, sl] = buf[e, sl] * nsp

        def grp(g, _):
            p = lax.rem(g, 2)
            idx_wait(g, p)
            idx_fire(jnp.minimum(g + 1, G - 1), 1 - p)
            gs = [pltpu.async_copy(y_sh.at[srcb.at[p * _NBUF + b]], rows[b],
                                   gsems[b])
                  for b in range(_NBUF)]
            ss = []
            for b in range(_NBUF):
                gs[b].wait()
                scale(rows[b], p * _NBUF + b)
                ss.append(pltpu.async_copy(rows[b],
                                           acc_sh.at[dstb.at[p * _NBUF + b]],
                                           ssems[b], add=True))
            for s in ss:
                s.wait()
            return 0
        lax.fori_loop(0, G, grp, 0)
        idx_wait(G - 1, 1 - lax.rem(G - 1, 2))
        plsc.subcore_barrier()

        pltpu.sync_copy(acc_sh.at[pl.ds(base, ROWS_PER_TILE)],
                        out_hbm.at[cid, pl.ds(base, ROWS_PER_TILE)])

    f = pl.kernel(
        body,
        out_type=jax.ShapeDtypeStruct((NC, NPAD, width), jnp.float32),
        mesh=_MESH,
        scratch_types=[
            pltpu.VMEM((2 * _NBUF, BATCH), jnp.int32),
            pltpu.VMEM((2 * _NBUF, BATCH), jnp.int32),
            pltpu.VMEM((2 * _NBUF, BATCH), jnp.float32),
            pltpu.VMEM((BATCH, width), jnp.float32),
            pltpu.VMEM((BATCH, width), jnp.float32),
            pltpu.VMEM((BATCH, width), jnp.float32),
            pltpu.VMEM((BATCH, width), jnp.float32),
            pltpu.VMEM_SHARED((NPAD, width), jnp.float32),
            pltpu.VMEM_SHARED((NPAD, width), jnp.float32),
            pltpu.SemaphoreType.DMA,
            pltpu.SemaphoreType.DMA,
            pltpu.SemaphoreType.DMA,
            pltpu.SemaphoreType.DMA,
            pltpu.SemaphoreType.DMA,
            pltpu.SemaphoreType.DMA,
            pltpu.SemaphoreType.DMA,
            pltpu.SemaphoreType.DMA,
            pltpu.SemaphoreType.DMA,
        ],
        compiler_params=_SC_PARAMS,
        name=f"cheby_prop_sc_{width}",
    )
    return f(y, srcp, dstp, normp)


_BLK = 1024


def _mm_tc(h, wcat):
    din = h.shape[1]
    wd = wcat.shape[1] // 3

    def body(h_ref, w_ref, o0, o1, o2):
        acc = jnp.dot(h_ref[...], w_ref[...], preferred_element_type=jnp.float32)
        o0[...] = acc[:, :wd]
        o1[...] = acc[:, wd:2 * wd]
        o2[...] = acc[:, 2 * wd:]

    outs = [jax.ShapeDtypeStruct((NPAD, wd), jnp.float32)] * 3
    return pl.pallas_call(
        body,
        grid=(NPAD // _BLK,),
        in_specs=[
            pl.BlockSpec((_BLK, din), lambda i: (i, 0)),
            pl.BlockSpec((din, 3 * wd), lambda i: (0, 0)),
        ],
        out_specs=[pl.BlockSpec((_BLK, wd), lambda i: (i, 0))] * 3,
        out_shape=outs,
    )(h, wcat)


def _comb_tc(p1, la):
    wd = p1.shape[1]

    def body(p1_ref, la0_ref, la1_ref, o_ref):
        o_ref[...] = p1_ref[...] + 2.0 * (la0_ref[0] + la1_ref[0])

    return pl.pallas_call(
        body,
        grid=(NPAD // _BLK,),
        in_specs=[
            pl.BlockSpec((_BLK, wd), lambda i: (i, 0)),
            pl.BlockSpec((1, _BLK, wd), lambda i: (0, i, 0)),
            pl.BlockSpec((1, _BLK, wd), lambda i: (1, i, 0)),
        ],
        out_specs=pl.BlockSpec((_BLK, wd), lambda i: (i, 0)),
        out_shape=jax.ShapeDtypeStruct((NPAD, wd), jnp.float32),
    )(p1, la, la)


def _end_mm_tc(p0, p2, lb, b, wcat):
    wd = p0.shape[1]
    wo = wcat.shape[1] // 3

    def body(p0_ref, p2_ref, lb0_ref, lb1_ref, b_ref, w_ref, o0, o1, o2):
        h = p0_ref[...] - p2_ref[...] + lb0_ref[0] + lb1_ref[0] + b_ref[...]
        h = jnp.maximum(h, 0.0)
        acc = jnp.dot(h, w_ref[...], preferred_element_type=jnp.float32)
        o0[...] = acc[:, :wo]
        o1[...] = acc[:, wo:2 * wo]
        o2[...] = acc[:, 2 * wo:]

    outs = [jax.ShapeDtypeStruct((NPAD, wo), jnp.float32)] * 3
    return pl.pallas_call(
        body,
        grid=(NPAD // _BLK,),
        in_specs=[
            pl.BlockSpec((_BLK, wd), lambda i: (i, 0)),
            pl.BlockSpec((_BLK, wd), lambda i: (i, 0)),
            pl.BlockSpec((1, _BLK, wd), lambda i: (0, i, 0)),
            pl.BlockSpec((1, _BLK, wd), lambda i: (1, i, 0)),
            pl.BlockSpec((1, wd), lambda i: (0, 0)),
            pl.BlockSpec((wd, 3 * wo), lambda i: (0, 0)),
        ],
        out_specs=[pl.BlockSpec((_BLK, wo), lambda i: (i, 0))] * 3,
        out_shape=outs,
    )(p0, p2, lb, lb, b.reshape(1, wd), wcat)


def _final_tc(p0, p2, lb, b):
    wd = p0.shape[1]

    def body(p0_ref, p2_ref, lb0_ref, lb1_ref, b_ref, o_ref):
        z = p0_ref[...] - p2_ref[...] + lb0_ref[0] + lb1_ref[0] + b_ref[...]
        z = jnp.maximum(z, 0.0)
        m = jnp.max(z, axis=-1, keepdims=True)
        zs = z - m
        lse = jnp.log(jnp.sum(jnp.exp(zs), axis=-1, keepdims=True))
        o_ref[...] = zs - lse

    return pl.pallas_call(
        body,
        grid=(NPAD // _BLK,),
        in_specs=[
            pl.BlockSpec((_BLK, wd), lambda i: (i, 0)),
            pl.BlockSpec((_BLK, wd), lambda i: (i, 0)),
            pl.BlockSpec((1, _BLK, wd), lambda i: (0, i, 0)),
            pl.BlockSpec((1, _BLK, wd), lambda i: (1, i, 0)),
            pl.BlockSpec((1, wd), lambda i: (0, 0)),
        ],
        out_specs=pl.BlockSpec((_BLK, wd), lambda i: (i, 0)),
        out_shape=jax.ShapeDtypeStruct((NPAD, wd), jnp.float32),
    )(p0, p2, lb, lb, b.reshape(1, wd))


def kernel(x, edge_index, edge_attr, W0, b0, W1, b1, W2, b2):
    src = edge_index[0]
    dst = edge_index[1]

    pad = EP - E
    srcp = jnp.concatenate([src, jnp.zeros((pad,), jnp.int32)]).reshape(NW, NB, BATCH)
    dstp = jnp.concatenate([dst, jnp.zeros((pad,), jnp.int32)]).reshape(NW, NB, BATCH)
    eap = jnp.concatenate([edge_attr, jnp.zeros((pad,), jnp.float32)]).reshape(NW, NB, BATCH)

    normp = _norm_sc(srcp, dstp, eap)

    wc1 = jnp.concatenate([W0[0], W0[1], W0[2]], axis=1)
    wc2 = jnp.concatenate([W1[0], W1[1], W1[2]], axis=1)
    wc3 = jnp.concatenate([W2[0], W2[1], W2[2]], axis=1)

    p0, p1, p2 = _mm_tc(x, wc1)
    la = _prop_sc(p2, srcp, dstp, normp, 64)
    bmat = _comb_tc(p1, la)
    lb = _prop_sc(bmat, srcp, dstp, normp, 64)
    q0, q1, q2 = _end_mm_tc(p0, p2, lb, b0, wc2)
    la = _prop_sc(q2, srcp, dstp, normp, 64)
    bmat = _comb_tc(q1, la)
    lb = _prop_sc(bmat, srcp, dstp, normp, 64)
    r0, r1, r2 = _end_mm_tc(q0, q2, lb, b1, wc3)
    la = _prop_sc(r2, srcp, dstp, normp, 32)
    bmat = _comb_tc(r1, la)
    lb = _prop_sc(bmat, srcp, dstp, normp, 32)

    return _final_tc(r0, r2, lb, b2)[:N]
